# Initial kernel scaffold; baseline (speedup 1.0000x reference)
import jax
import jax.numpy as jnp
from jax.experimental import pallas as pl


def kernel(x, x_cov, edge_index, edge_weight, norm_weight, bn_gamma, bn_beta, p0_W, p0_C, p1_W, p1_C, feature_corr, mlp_W1, mlp_b1, mlp_a1, mlp_W2, mlp_b2, mlp_a2, mlp_W3, mlp_b3):
    raise NotImplementedError("write your pallas kernel here")


# speedup vs baseline: 2.3859x; 2.3859x over previous
import functools

import jax
import jax.numpy as jnp
from jax import lax
from jax.experimental import pallas as pl
from jax.experimental.pallas import tpu as pltpu
from jax.experimental.pallas import tpu_sc as plsc

N = 10000
NP = 10240
D = 128
DH = D // 2
K = 10
NTILE = 16
RPT = NP // NTILE
CHUNK = 128


def _bn_body(x_ref, g_ref, b_ref, lo_ref, hi_ref):
    x = x_ref[...]
    mu = jnp.mean(x, axis=0, keepdims=True)
    xc = x - mu
    var = jnp.mean(xc * xc, axis=0, keepdims=True)
    xb = xc * (g_ref[...] / jnp.sqrt(var + 1e-5)) + b_ref[...]
    zpad = jnp.zeros((NP - N, D), jnp.float32)
    xbp = jnp.concatenate([xb, zpad], axis=0)
    lo_ref[...] = xbp[:, :DH]
    hi_ref[...] = xbp[:, DH:]


def _batchnorm(x, gamma, beta):
    return pl.pallas_call(
        _bn_body,
        out_shape=(
            jax.ShapeDtypeStruct((NP, DH), jnp.float32),
            jax.ShapeDtypeStruct((NP, DH), jnp.float32),
        ),
    )(x, gamma.reshape(1, D), beta.reshape(1, D))


def _mlp_body(lo_ref, hi_ref, w1a_ref, w1b_ref, b1_ref, a1_ref,
              w2_ref, b2_ref, a2_ref, w3_ref, b3_ref, out_ref):
    f32 = jnp.float32
    h1 = (lax.dot_general(lo_ref[...], w1a_ref[...], (((1,), (0,)), ((), ())),
                          preferred_element_type=f32)
          + lax.dot_general(hi_ref[...], w1b_ref[...], (((1,), (0,)), ((), ())),
                            preferred_element_type=f32)
          + b1_ref[...])
    a1 = a1_ref[0, 0]
    h1 = jnp.where(h1 >= 0, h1, a1 * h1)
    h2 = lax.dot_general(h1, w2_ref[...], (((1,), (0,)), ((), ())),
                         preferred_element_type=f32) + b2_ref[...]
    a2 = a2_ref[0, 0]
    h2 = jnp.where(h2 >= 0, h2, a2 * h2)
    logits = lax.dot_general(h2, w3_ref[...], (((1,), (0,)), ((), ())),
                             preferred_element_type=f32) + b3_ref[...]
    lane = lax.broadcasted_iota(jnp.int32, logits.shape, 1)
    neg = jnp.float32(-1e30)
    masked = jnp.where(lane < 2, logits, neg)
    m = jnp.max(masked, axis=1, keepdims=True)
    e = jnp.where(lane < 2, jnp.exp(masked - m), 0.0)
    lse = m + jnp.log(jnp.sum(e, axis=1, keepdims=True))
    res = logits - lse
    out_ref[...] = res[:, :2]


def _mlp(lo, hi, w1, b1, a1, w2, b2, a2, w3, b3):
    w3p = jnp.zeros((64, 128), jnp.float32).at[:, :2].set(w3)
    b3p = jnp.zeros((1, 128), jnp.float32).at[0, :2].set(b3)
    return pl.pallas_call(
        _mlp_body,
        out_shape=jax.ShapeDtypeStruct((N, 2), jnp.float32),
    )(lo, hi, w1[:DH], w1[DH:], b1.reshape(1, 64), a1.reshape(1, 1),
      w2, b2.reshape(1, 64), a2.reshape(1, 1), w3p, b3p)


def _prop_tile(x_in, work, zeros_hbm, cols_v, rows_v, w_v, g, y_sp, sem, s):
    nch = cols_v.shape[0]
    rs = pl.ds(s * RPT, RPT)

    for k in range(RPT // CHUNK):
        part = pl.ds(s * RPT + k * CHUNK, CHUNK)
        pltpu.sync_copy(x_in.at[part], g)
        pltpu.sync_copy(g, work.at[part])
    plsc.subcore_barrier()

    @pl.loop(0, K)
    def _iter(_):
        pltpu.sync_copy(zeros_hbm.at[rs], y_sp.at[rs])
        plsc.subcore_barrier()

        @pl.loop(0, nch)
        def _chunk(ch):
            pltpu.async_copy(work.at[cols_v.at[ch]], g, sem).wait()

            @pl.loop(0, CHUNK // 16)
            def _grp(g16):
                wv16 = w_v[ch, pl.ds(g16 * 16, 16)]
                for e in range(16):
                    w1 = jnp.full((16,), wv16[e], jnp.float32)
                    j = g16 * 16 + e
                    for q in range(DH // 16):
                        sl = pl.ds(q * 16, 16)
                        g[j, sl] = g[j, sl] * w1

            pltpu.sync_copy(g, y_sp.at[rows_v.at[ch]], add=True)

        plsc.subcore_barrier()
        pltpu.sync_copy(y_sp.at[rs], work.at[rs])
        plsc.subcore_barrier()


def _propagate(xb_lo, xb_hi, cols3, rows3, w3, zeros):
    nch = cols3.shape[1]
    mesh = plsc.VectorSubcoreMesh(core_axis_name="c", subcore_axis_name="s")

    @functools.partial(
        pl.kernel,
        out_type=(
            jax.ShapeDtypeStruct((NP, DH), jnp.float32),
            jax.ShapeDtypeStruct((NP, DH), jnp.float32),
        ),
        mesh=mesh,
        compiler_params=pltpu.CompilerParams(use_tc_tiling_on_sc=False),
        scratch_types=[
            pltpu.VMEM((nch, CHUNK), jnp.int32),
            pltpu.VMEM((nch, CHUNK), jnp.int32),
            pltpu.VMEM((nch, CHUNK), jnp.float32),
            pltpu.VMEM((CHUNK, DH), jnp.float32),
            pltpu.VMEM_SHARED((NP, DH), jnp.float32),
            pltpu.SemaphoreType.DMA,
        ],
    )
    def prop(xlo, xhi, cols_hbm, rows_hbm, w_hbm, zeros_hbm,
             out_lo, out_hi, cols_v, rows_v, w_v, g, y_sp, sem):
        c = lax.axis_index("c")
        s = lax.axis_index("s")
        pltpu.sync_copy(cols_hbm.at[s], cols_v)
        pltpu.sync_copy(rows_hbm.at[s], rows_v)
        pltpu.sync_copy(w_hbm.at[s], w_v)

        @pl.when(c == 0)
        def _():
            _prop_tile(xlo, out_lo, zeros_hbm, cols_v, rows_v, w_v, g, y_sp,
                       sem, s)

        @pl.when(c == 1)
        def _():
            _prop_tile(xhi, out_hi, zeros_hbm, cols_v, rows_v, w_v, g, y_sp,
                       sem, s)

    return prop(xb_lo, xb_hi, cols3, rows3, w3, zeros)


def kernel(x, x_cov, edge_index, edge_weight, norm_weight, bn_gamma, bn_beta,
           p0_W, p0_C, p1_W, p1_C, feature_corr,
           mlp_W1, mlp_b1, mlp_a1, mlp_W2, mlp_b2, mlp_a2, mlp_W3, mlp_b3):
    E = edge_index.shape[1]
    ept = ((E + NTILE * CHUNK - 1) // (NTILE * CHUNK)) * CHUNK
    epad = NTILE * ept
    pad = epad - E
    rows = jnp.concatenate([edge_index[0], jnp.zeros((pad,), edge_index.dtype)])
    cols = jnp.concatenate([edge_index[1], jnp.zeros((pad,), edge_index.dtype)])
    w = jnp.concatenate([norm_weight, jnp.zeros((pad,), norm_weight.dtype)])
    nch = ept // CHUNK
    rows3 = rows.reshape(NTILE, nch, CHUNK).astype(jnp.int32)
    cols3 = cols.reshape(NTILE, nch, CHUNK).astype(jnp.int32)
    w3 = w.reshape(NTILE, nch, CHUNK)
    zeros = jnp.zeros((NP, DH), jnp.float32)

    xb_lo, xb_hi = _batchnorm(x, bn_gamma, bn_beta)
    xh_lo, xh_hi = _propagate(xb_lo, xb_hi, cols3, rows3, w3, zeros)
    return _mlp(xh_lo[:N], xh_hi[:N], mlp_W1, mlp_b1, mlp_a1,
                mlp_W2, mlp_b2, mlp_a2, mlp_W3, mlp_b3)


# speedup vs baseline: 3.2700x; 1.3705x over previous
import functools

import jax
import jax.numpy as jnp
from jax import lax
from jax.experimental import pallas as pl
from jax.experimental.pallas import tpu as pltpu
from jax.experimental.pallas import tpu_sc as plsc

N = 10000
NP = 10240
D = 128
DH = D // 2
K = 10
NTILE = 16
RPT = NP // NTILE
CHUNK = 128


def _bn_body(x_ref, g_ref, b_ref, lo_ref, hi_ref):
    x = x_ref[...]
    mu = jnp.mean(x, axis=0, keepdims=True)
    xc = x - mu
    var = jnp.mean(xc * xc, axis=0, keepdims=True)
    xb = xc * (g_ref[...] / jnp.sqrt(var + 1e-5)) + b_ref[...]
    zpad = jnp.zeros((NP - N, D), jnp.float32)
    xbp = jnp.concatenate([xb, zpad], axis=0)
    lo_ref[...] = xbp[:, :DH]
    hi_ref[...] = xbp[:, DH:]


def _batchnorm(x, gamma, beta):
    return pl.pallas_call(
        _bn_body,
        out_shape=(
            jax.ShapeDtypeStruct((NP, DH), jnp.float32),
            jax.ShapeDtypeStruct((NP, DH), jnp.float32),
        ),
    )(x, gamma.reshape(1, D), beta.reshape(1, D))


def _mlp_body(lo_ref, hi_ref, w1a_ref, w1b_ref, b1_ref, a1_ref,
              w2_ref, b2_ref, a2_ref, w3_ref, b3_ref, out_ref):
    f32 = jnp.float32
    h1 = (lax.dot_general(lo_ref[...], w1a_ref[...], (((1,), (0,)), ((), ())),
                          preferred_element_type=f32)
          + lax.dot_general(hi_ref[...], w1b_ref[...], (((1,), (0,)), ((), ())),
                            preferred_element_type=f32)
          + b1_ref[...])
    a1 = a1_ref[0, 0]
    h1 = jnp.where(h1 >= 0, h1, a1 * h1)
    h2 = lax.dot_general(h1, w2_ref[...], (((1,), (0,)), ((), ())),
                         preferred_element_type=f32) + b2_ref[...]
    a2 = a2_ref[0, 0]
    h2 = jnp.where(h2 >= 0, h2, a2 * h2)
    logits = lax.dot_general(h2, w3_ref[...], (((1,), (0,)), ((), ())),
                             preferred_element_type=f32) + b3_ref[...]
    lane = lax.broadcasted_iota(jnp.int32, logits.shape, 1)
    neg = jnp.float32(-1e30)
    masked = jnp.where(lane < 2, logits, neg)
    m = jnp.max(masked, axis=1, keepdims=True)
    e = jnp.where(lane < 2, jnp.exp(masked - m), 0.0)
    lse = m + jnp.log(jnp.sum(e, axis=1, keepdims=True))
    res = logits - lse
    out_ref[...] = res[:, :2]


def _mlp(lo, hi, w1, b1, a1, w2, b2, a2, w3, b3):
    w3p = jnp.zeros((64, 128), jnp.float32).at[:, :2].set(w3)
    b3p = jnp.zeros((1, 128), jnp.float32).at[0, :2].set(b3)
    return pl.pallas_call(
        _mlp_body,
        out_shape=jax.ShapeDtypeStruct((N, 2), jnp.float32),
    )(lo, hi, w1[:DH], w1[DH:], b1.reshape(1, 64), a1.reshape(1, 1),
      w2, b2.reshape(1, 64), a2.reshape(1, 1), w3p, b3p)


def _scale(g, s_buf, w_v, ch):
    @pl.loop(0, CHUNK // 16)
    def _grp(g16):
        wv16 = w_v[ch, pl.ds(g16 * 16, 16)]
        for e in range(16):
            w1 = jnp.full((16,), wv16[e], jnp.float32)
            j = g16 * 16 + e
            for q in range(DH // 16):
                sl = pl.ds(q * 16, 16)
                s_buf[j, sl] = g[j, sl] * w1


def _prop_tile(x_in, work, zeros_hbm, cols_v, rows_v, w_v,
               g0, g1, s0, s1, y_sp, gsem0, gsem1, ssem0, ssem1, s):
    nch = rows_v.shape[0]
    rs = pl.ds(s * RPT, RPT)

    for k in range(RPT // CHUNK):
        part = pl.ds(s * RPT + k * CHUNK, CHUNK)
        pltpu.sync_copy(x_in.at[part], g0)
        pltpu.sync_copy(g0, work.at[part])
    plsc.subcore_barrier()

    @pl.loop(0, K)
    def _iter(_):
        pltpu.sync_copy(zeros_hbm.at[rs], y_sp.at[rs])
        plsc.subcore_barrier()

        pltpu.async_copy(work.at[cols_v.at[0]], g0, gsem0)
        pltpu.async_copy(work.at[cols_v.at[1]], g1, gsem1)

        @pl.loop(0, nch, step=2)
        def _pair(ch):
            for par, g, sb, gsem, ssem in ((0, g0, s0, gsem0, ssem0),
                                           (1, g1, s1, gsem1, ssem1)):
                c = ch + par
                pltpu.make_async_copy(work.at[cols_v.at[c]], g, gsem).wait()

                @pl.when(c >= 2)
                def _():
                    pltpu.make_async_copy(
                        sb, y_sp.at[rows_v.at[c - 2]], ssem).wait()

                _scale(g, sb, w_v, c)
                pltpu.async_copy(sb, y_sp.at[rows_v.at[c]], ssem, add=True)
                pltpu.async_copy(work.at[cols_v.at[c + 2]], g, gsem)

        pltpu.make_async_copy(work.at[cols_v.at[nch]], g0, gsem0).wait()
        pltpu.make_async_copy(work.at[cols_v.at[nch + 1]], g1, gsem1).wait()
        pltpu.make_async_copy(s0, y_sp.at[rows_v.at[nch - 2]], ssem0).wait()
        pltpu.make_async_copy(s1, y_sp.at[rows_v.at[nch - 1]], ssem1).wait()

        plsc.subcore_barrier()
        pltpu.sync_copy(y_sp.at[rs], work.at[rs])
        plsc.subcore_barrier()


def _propagate(xb_lo, xb_hi, cols3, rows3, w3, zeros):
    nch = rows3.shape[1]
    mesh = plsc.VectorSubcoreMesh(core_axis_name="c", subcore_axis_name="s")

    @functools.partial(
        pl.kernel,
        out_type=(
            jax.ShapeDtypeStruct((NP, DH), jnp.float32),
            jax.ShapeDtypeStruct((NP, DH), jnp.float32),
        ),
        mesh=mesh,
        compiler_params=pltpu.CompilerParams(use_tc_tiling_on_sc=False),
        scratch_types=[
            pltpu.VMEM((nch + 2, CHUNK), jnp.int32),
            pltpu.VMEM((nch, CHUNK), jnp.int32),
            pltpu.VMEM((nch, CHUNK), jnp.float32),
            pltpu.VMEM((CHUNK, DH), jnp.float32),
            pltpu.VMEM((CHUNK, DH), jnp.float32),
            pltpu.VMEM((CHUNK, DH), jnp.float32),
            pltpu.VMEM((CHUNK, DH), jnp.float32),
            pltpu.VMEM_SHARED((NP, DH), jnp.float32),
            pltpu.SemaphoreType.DMA,
            pltpu.SemaphoreType.DMA,
            pltpu.SemaphoreType.DMA,
            pltpu.SemaphoreType.DMA,
        ],
    )
    def prop(xlo, xhi, cols_hbm, rows_hbm, w_hbm, zeros_hbm,
             out_lo, out_hi, cols_v, rows_v, w_v, g0, g1, s0, s1, y_sp,
             gsem0, gsem1, ssem0, ssem1):
        c = lax.axis_index("c")
        s = lax.axis_index("s")
        pltpu.sync_copy(cols_hbm.at[s], cols_v)
        pltpu.sync_copy(rows_hbm.at[s], rows_v)
        pltpu.sync_copy(w_hbm.at[s], w_v)

        @pl.when(c == 0)
        def _():
            _prop_tile(xlo, out_lo, zeros_hbm, cols_v, rows_v, w_v,
                       g0, g1, s0, s1, y_sp, gsem0, gsem1, ssem0, ssem1, s)

        @pl.when(c == 1)
        def _():
            _prop_tile(xhi, out_hi, zeros_hbm, cols_v, rows_v, w_v,
                       g0, g1, s0, s1, y_sp, gsem0, gsem1, ssem0, ssem1, s)

    return prop(xb_lo, xb_hi, cols3, rows3, w3, zeros)


def kernel(x, x_cov, edge_index, edge_weight, norm_weight, bn_gamma, bn_beta,
           p0_W, p0_C, p1_W, p1_C, feature_corr,
           mlp_W1, mlp_b1, mlp_a1, mlp_W2, mlp_b2, mlp_a2, mlp_W3, mlp_b3):
    E = edge_index.shape[1]
    quant = 2 * CHUNK
    ept = ((E + NTILE * quant - 1) // (NTILE * quant)) * quant
    epad = NTILE * ept
    pad = epad - E
    rows = jnp.concatenate([edge_index[0], jnp.zeros((pad,), edge_index.dtype)])
    cols = jnp.concatenate([edge_index[1], jnp.zeros((pad,), edge_index.dtype)])
    w = jnp.concatenate([norm_weight, jnp.zeros((pad,), norm_weight.dtype)])
    nch = ept // CHUNK
    rows3 = rows.reshape(NTILE, nch, CHUNK).astype(jnp.int32)
    cols3 = cols.reshape(NTILE, nch, CHUNK).astype(jnp.int32)
    cols3 = jnp.pad(cols3, ((0, 0), (0, 2), (0, 0)))
    w3 = w.reshape(NTILE, nch, CHUNK)
    zeros = jnp.zeros((NP, DH), jnp.float32)

    xb_lo, xb_hi = _batchnorm(x, bn_gamma, bn_beta)
    xh_lo, xh_hi = _propagate(xb_lo, xb_hi, cols3, rows3, w3, zeros)
    return _mlp(xh_lo[:N], xh_hi[:N], mlp_W1, mlp_b1, mlp_a1,
                mlp_W2, mlp_b2, mlp_a2, mlp_W3, mlp_b3)
